# Initial kernel scaffold; baseline (speedup 1.0000x reference)
import jax
import jax.numpy as jnp
from jax.experimental import pallas as pl


def kernel(TRAIN, x, edge_index, W1, b1, W2, b2, W3, b3, Wf1, bf1, Wf2, bf2, Wf3, bf3):
    raise NotImplementedError("write your pallas kernel here")


# speedup vs baseline: 15.4346x; 15.4346x over previous
import functools

import jax
import jax.numpy as jnp
from jax import lax
from jax.experimental import pallas as pl
from jax.experimental.pallas import tpu as pltpu
from jax.experimental.pallas import tpu_sc as plsc

N = 10000
P = 10240
E = 320000
D = 128
C = 16
NC, NS = 2, 16
NW = NC * NS
CHUNK = 80
EPT = E // NW
NCH = EPT // CHUNK
ROWS_T = P // NS

_mesh = plsc.VectorSubcoreMesh(
    core_axis_name="c", subcore_axis_name="s", num_cores=NC, num_subcores=NS)


def _deg_body(dst3, ones_h, zeros_h, deg_out, didx, ones_v, dacc):
    cid = lax.axis_index("c")
    sid = lax.axis_index("s")
    wid = cid * NS + sid

    pltpu.sync_copy(dst3.at[wid], didx)
    pltpu.sync_copy(ones_h, ones_v)
    pltpu.sync_copy(zeros_h.at[pl.ds(sid * ROWS_T, ROWS_T)],
                    dacc.at[pl.ds(sid * ROWS_T, ROWS_T)])
    plsc.subcore_barrier()

    def body(j, _):
        pltpu.sync_copy(ones_v, dacc.at[didx.at[j]], add=True)
        return 0

    lax.fori_loop(0, NCH, body, 0)
    plsc.subcore_barrier()
    pltpu.sync_copy(dacc.at[pl.ds(sid * ROWS_T, ROWS_T)],
                    deg_out.at[cid, pl.ds(sid * ROWS_T, ROWS_T)])


_deg_call = pl.kernel(
    _deg_body,
    out_type=jax.ShapeDtypeStruct((NC, P, D), jnp.float32),
    mesh=_mesh,
    scratch_types=[
        pltpu.VMEM((NCH, CHUNK), jnp.int32),
        pltpu.VMEM((CHUNK, D), jnp.float32),
        pltpu.VMEM_SHARED((P, D), jnp.float32),
    ],
)


def _spmm_body(hs, src3, dst3, zeros_h, acc_out, sidx, didx, rows, acc):
    cid = lax.axis_index("c")
    sid = lax.axis_index("s")
    wid = cid * NS + sid

    pltpu.sync_copy(src3.at[wid], sidx)
    pltpu.sync_copy(dst3.at[wid], didx)
    pltpu.sync_copy(zeros_h.at[pl.ds(sid * ROWS_T, ROWS_T)],
                    acc.at[pl.ds(sid * ROWS_T, ROWS_T)])
    plsc.subcore_barrier()

    def body(j, _):
        pltpu.sync_copy(hs.at[sidx.at[j]], rows)
        pltpu.sync_copy(rows, acc.at[didx.at[j]], add=True)
        return 0

    lax.fori_loop(0, NCH, body, 0)
    plsc.subcore_barrier()
    pltpu.sync_copy(acc.at[pl.ds(sid * ROWS_T, ROWS_T)],
                    acc_out.at[cid, pl.ds(sid * ROWS_T, ROWS_T)])


_spmm_call = pl.kernel(
    _spmm_body,
    out_type=jax.ShapeDtypeStruct((NC, P, D), jnp.float32),
    mesh=_mesh,
    scratch_types=[
        pltpu.VMEM((NCH, CHUNK), jnp.int32),
        pltpu.VMEM((NCH, CHUNK), jnp.int32),
        pltpu.VMEM((CHUNK, D), jnp.float32),
        pltpu.VMEM_SHARED((P, D), jnp.float32),
    ],
)


BLK = 1024
GRID = P // BLK


def _first_body(x_ref, w_ref, deg_ref, hs_ref, dinv_ref):
    dinv = lax.rsqrt(deg_ref[0][:, 0:1] + deg_ref[1][:, 0:1] + 1.0)
    h = jnp.dot(x_ref[...], w_ref[...], preferred_element_type=jnp.float32)
    hs_ref[...] = h * dinv
    dinv_ref[...] = dinv


def _tc_first(xp, w1, deg):
    return pl.pallas_call(
        _first_body,
        grid=(GRID,),
        in_specs=[
            pl.BlockSpec((BLK, D), lambda i: (i, 0)),
            pl.BlockSpec((D, D), lambda i: (0, 0)),
            pl.BlockSpec((NC, BLK, D), lambda i: (0, i, 0)),
        ],
        out_specs=[
            pl.BlockSpec((BLK, D), lambda i: (i, 0)),
            pl.BlockSpec((BLK, 1), lambda i: (i, 0)),
        ],
        out_shape=[
            jax.ShapeDtypeStruct((P, D), jnp.float32),
            jax.ShapeDtypeStruct((P, 1), jnp.float32),
        ],
    )(xp, w1, deg)


def _mid_body(acc_ref, hsp_ref, dinv_ref, b_ref, w_ref, out_ref):
    t = acc_ref[0] + acc_ref[1] + hsp_ref[...]
    y = jnp.maximum(t * dinv_ref[...] + b_ref[...], 0.0)
    out_ref[...] = jnp.dot(
        y, w_ref[...], preferred_element_type=jnp.float32) * dinv_ref[...]


def _tc_mid(accp, hsp, dinv, b_prev, w_next):
    return pl.pallas_call(
        _mid_body,
        grid=(GRID,),
        in_specs=[
            pl.BlockSpec((NC, BLK, D), lambda i: (0, i, 0)),
            pl.BlockSpec((BLK, D), lambda i: (i, 0)),
            pl.BlockSpec((BLK, 1), lambda i: (i, 0)),
            pl.BlockSpec((1, D), lambda i: (0, 0)),
            pl.BlockSpec((D, D), lambda i: (0, 0)),
        ],
        out_specs=pl.BlockSpec((BLK, D), lambda i: (i, 0)),
        out_shape=jax.ShapeDtypeStruct((P, D), jnp.float32),
    )(accp, hsp, dinv, b_prev, w_next)


def _head_body(acc_ref, hsp_ref, dinv_ref, b3_ref, wf1_ref, bf1_ref,
               wf2_ref, bf2_ref, wf3_ref, bf3_ref, out_ref):
    t = acc_ref[0] + acc_ref[1] + hsp_ref[...]
    y = jnp.maximum(t * dinv_ref[...] + b3_ref[...], 0.0)
    y = jnp.maximum(
        jnp.dot(y, wf1_ref[...], preferred_element_type=jnp.float32)
        + bf1_ref[...], 0.0)
    y = jnp.maximum(
        jnp.dot(y, wf2_ref[...], preferred_element_type=jnp.float32)
        + bf2_ref[...], 0.0)
    z = jnp.dot(y, wf3_ref[...], preferred_element_type=jnp.float32) + bf3_ref[...]
    z = z - jnp.max(z, axis=1, keepdims=True)
    out_ref[...] = z - jnp.log(jnp.sum(jnp.exp(z), axis=1, keepdims=True))


def _tc_head(accp, hsp, dinv, b3, wf1, bf1, wf2, bf2, wf3, bf3):
    return pl.pallas_call(
        _head_body,
        grid=(GRID,),
        in_specs=[
            pl.BlockSpec((NC, BLK, D), lambda i: (0, i, 0)),
            pl.BlockSpec((BLK, D), lambda i: (i, 0)),
            pl.BlockSpec((BLK, 1), lambda i: (i, 0)),
            pl.BlockSpec((1, D), lambda i: (0, 0)),
            pl.BlockSpec((D, D), lambda i: (0, 0)),
            pl.BlockSpec((1, D), lambda i: (0, 0)),
            pl.BlockSpec((D, D), lambda i: (0, 0)),
            pl.BlockSpec((1, D), lambda i: (0, 0)),
            pl.BlockSpec((D, C), lambda i: (0, 0)),
            pl.BlockSpec((1, C), lambda i: (0, 0)),
        ],
        out_specs=pl.BlockSpec((BLK, C), lambda i: (i, 0)),
        out_shape=jax.ShapeDtypeStruct((P, C), jnp.float32),
    )(accp, hsp, dinv, b3, wf1, bf1, wf2, bf2, wf3, bf3)


def kernel(TRAIN, x, edge_index, W1, b1, W2, b2, W3, b3,
           Wf1, bf1, Wf2, bf2, Wf3, bf3):
    del TRAIN
    src = edge_index[0].astype(jnp.int32)
    dst = edge_index[1].astype(jnp.int32)
    src3 = src.reshape(NW, NCH, CHUNK)
    dst3 = dst.reshape(NW, NCH, CHUNK)

    xp = jnp.zeros((P, D), jnp.float32).at[:N].set(x)
    zeros_pd = jnp.zeros((P, D), jnp.float32)
    ones_c = jnp.ones((CHUNK, D), jnp.float32)

    deg = _deg_call(dst3, ones_c, zeros_pd)
    hs1, dinv = _tc_first(xp, W1, deg)
    acc1 = _spmm_call(hs1, src3, dst3, zeros_pd)
    hs2 = _tc_mid(acc1, hs1, dinv, b1.reshape(1, D), W2)
    acc2 = _spmm_call(hs2, src3, dst3, zeros_pd)
    hs3 = _tc_mid(acc2, hs2, dinv, b2.reshape(1, D), W3)
    acc3 = _spmm_call(hs3, src3, dst3, zeros_pd)
    out = _tc_head(acc3, hs3, dinv, b3.reshape(1, D),
                   Wf1, bf1.reshape(1, D), Wf2, bf2.reshape(1, D),
                   Wf3, bf3.reshape(1, C))
    return out[:N]


# speedup vs baseline: 19.2363x; 1.2463x over previous
import functools

import jax
import jax.numpy as jnp
from jax import lax
from jax.experimental import pallas as pl
from jax.experimental.pallas import tpu as pltpu
from jax.experimental.pallas import tpu_sc as plsc

N = 10000
P = 10240
E = 320000
D = 128
C = 16
NC, NS = 2, 16
NW = NC * NS
CHUNK = 80
EPT = E // NW
NCH = EPT // CHUNK
ROWS_T = P // NS

_mesh = plsc.VectorSubcoreMesh(
    core_axis_name="c", subcore_axis_name="s", num_cores=NC, num_subcores=NS)


DEG_INFLIGHT = 4


def _deg_body(dst3, ones_h, zeros_h, deg_out, didx, ones_v, dacc, ssem):
    cid = lax.axis_index("c")
    sid = lax.axis_index("s")
    wid = cid * NS + sid

    pltpu.sync_copy(dst3.at[wid], didx)
    pltpu.sync_copy(ones_h, ones_v)
    pltpu.sync_copy(zeros_h.at[pl.ds(sid * ROWS_T, ROWS_T)],
                    dacc.at[pl.ds(sid * ROWS_T, ROWS_T)])
    plsc.subcore_barrier()

    def body(j, _):
        pltpu.async_copy(ones_v, dacc.at[didx.at[j]], ssem, add=True)

        @pl.when(j >= DEG_INFLIGHT)
        def _():
            pltpu.make_async_copy(ones_v, dacc.at[didx.at[0]], ssem).wait()

        return 0

    lax.fori_loop(0, NCH, body, 0)

    def drain(j, _):
        pltpu.make_async_copy(ones_v, dacc.at[didx.at[0]], ssem).wait()
        return 0

    lax.fori_loop(0, DEG_INFLIGHT, drain, 0)
    plsc.subcore_barrier()
    pltpu.sync_copy(dacc.at[pl.ds(sid * ROWS_T, ROWS_T)],
                    deg_out.at[cid, pl.ds(sid * ROWS_T, ROWS_T)])


_deg_call = pl.kernel(
    _deg_body,
    out_type=jax.ShapeDtypeStruct((NC, P, D), jnp.float32),
    mesh=_mesh,
    scratch_types=[
        pltpu.VMEM((NCH, CHUNK), jnp.int32),
        pltpu.VMEM((CHUNK, D), jnp.float32),
        pltpu.VMEM_SHARED((P, D), jnp.float32),
        pltpu.SemaphoreType.DMA,
    ],
)


def _spmm_body(hs, src3, dst3, zeros_h, acc_out, sidx, didx, rows, acc,
               gsem, ssem, isem):
    cid = lax.axis_index("c")
    sid = lax.axis_index("s")
    wid = cid * NS + sid

    pltpu.sync_copy(src3.at[wid], sidx)
    pltpu.sync_copy(zeros_h.at[pl.ds(sid * ROWS_T, ROWS_T)],
                    acc.at[pl.ds(sid * ROWS_T, ROWS_T)])
    plsc.subcore_barrier()

    pltpu.sync_copy(dst3.at[wid, 0], didx.at[0])
    pltpu.async_copy(hs.at[sidx.at[0]], rows.at[0], gsem)

    def body(j, _):
        slot = lax.bitwise_and(j, 1)
        nslot = 1 - slot

        @pl.when(j < NCH - 1)
        def _():
            pltpu.async_copy(dst3.at[wid, j + 1], didx.at[nslot], isem)

        pltpu.make_async_copy(hs.at[sidx.at[j]], rows.at[slot], gsem).wait()

        @pl.when(j < NCH - 1)
        def _():
            pltpu.async_copy(hs.at[sidx.at[j + 1]], rows.at[nslot], gsem)

        pltpu.async_copy(rows.at[slot], acc.at[didx.at[slot]], ssem, add=True)

        @pl.when(j < NCH - 1)
        def _():
            pltpu.make_async_copy(dst3.at[wid, j + 1], didx.at[nslot],
                                  isem).wait()

        pltpu.make_async_copy(rows.at[slot], acc.at[didx.at[slot]], ssem).wait()
        return 0

    lax.fori_loop(0, NCH, body, 0)
    plsc.subcore_barrier()
    pltpu.sync_copy(acc.at[pl.ds(sid * ROWS_T, ROWS_T)],
                    acc_out.at[cid, pl.ds(sid * ROWS_T, ROWS_T)])


_spmm_call = pl.kernel(
    _spmm_body,
    out_type=jax.ShapeDtypeStruct((NC, P, D), jnp.float32),
    mesh=_mesh,
    scratch_types=[
        pltpu.VMEM((NCH, CHUNK), jnp.int32),
        pltpu.VMEM((2, CHUNK), jnp.int32),
        pltpu.VMEM((2, CHUNK, D), jnp.float32),
        pltpu.VMEM_SHARED((P, D), jnp.float32),
        pltpu.SemaphoreType.DMA,
        pltpu.SemaphoreType.DMA,
        pltpu.SemaphoreType.DMA,
    ],
)


BLK = 1024
GRID = P // BLK


def _first_body(x_ref, w_ref, deg_ref, hs_ref, dinv_ref):
    dinv = lax.rsqrt(deg_ref[0][:, 0:1] + deg_ref[1][:, 0:1] + 1.0)
    h = jnp.dot(x_ref[...], w_ref[...], preferred_element_type=jnp.float32)
    hs_ref[...] = h * dinv
    dinv_ref[...] = dinv


def _tc_first(xp, w1, deg):
    return pl.pallas_call(
        _first_body,
        grid=(GRID,),
        in_specs=[
            pl.BlockSpec((BLK, D), lambda i: (i, 0)),
            pl.BlockSpec((D, D), lambda i: (0, 0)),
            pl.BlockSpec((NC, BLK, D), lambda i: (0, i, 0)),
        ],
        out_specs=[
            pl.BlockSpec((BLK, D), lambda i: (i, 0)),
            pl.BlockSpec((BLK, 1), lambda i: (i, 0)),
        ],
        out_shape=[
            jax.ShapeDtypeStruct((P, D), jnp.float32),
            jax.ShapeDtypeStruct((P, 1), jnp.float32),
        ],
    )(xp, w1, deg)


def _mid_body(acc_ref, hsp_ref, dinv_ref, b_ref, w_ref, out_ref):
    t = acc_ref[0] + acc_ref[1] + hsp_ref[...]
    y = jnp.maximum(t * dinv_ref[...] + b_ref[...], 0.0)
    out_ref[...] = jnp.dot(
        y, w_ref[...], preferred_element_type=jnp.float32) * dinv_ref[...]


def _tc_mid(accp, hsp, dinv, b_prev, w_next):
    return pl.pallas_call(
        _mid_body,
        grid=(GRID,),
        in_specs=[
            pl.BlockSpec((NC, BLK, D), lambda i: (0, i, 0)),
            pl.BlockSpec((BLK, D), lambda i: (i, 0)),
            pl.BlockSpec((BLK, 1), lambda i: (i, 0)),
            pl.BlockSpec((1, D), lambda i: (0, 0)),
            pl.BlockSpec((D, D), lambda i: (0, 0)),
        ],
        out_specs=pl.BlockSpec((BLK, D), lambda i: (i, 0)),
        out_shape=jax.ShapeDtypeStruct((P, D), jnp.float32),
    )(accp, hsp, dinv, b_prev, w_next)


def _head_body(acc_ref, hsp_ref, dinv_ref, b3_ref, wf1_ref, bf1_ref,
               wf2_ref, bf2_ref, wf3_ref, bf3_ref, out_ref):
    t = acc_ref[0] + acc_ref[1] + hsp_ref[...]
    y = jnp.maximum(t * dinv_ref[...] + b3_ref[...], 0.0)
    y = jnp.maximum(
        jnp.dot(y, wf1_ref[...], preferred_element_type=jnp.float32)
        + bf1_ref[...], 0.0)
    y = jnp.maximum(
        jnp.dot(y, wf2_ref[...], preferred_element_type=jnp.float32)
        + bf2_ref[...], 0.0)
    z = jnp.dot(y, wf3_ref[...], preferred_element_type=jnp.float32) + bf3_ref[...]
    z = z - jnp.max(z, axis=1, keepdims=True)
    out_ref[...] = z - jnp.log(jnp.sum(jnp.exp(z), axis=1, keepdims=True))


def _tc_head(accp, hsp, dinv, b3, wf1, bf1, wf2, bf2, wf3, bf3):
    return pl.pallas_call(
        _head_body,
        grid=(GRID,),
        in_specs=[
            pl.BlockSpec((NC, BLK, D), lambda i: (0, i, 0)),
            pl.BlockSpec((BLK, D), lambda i: (i, 0)),
            pl.BlockSpec((BLK, 1), lambda i: (i, 0)),
            pl.BlockSpec((1, D), lambda i: (0, 0)),
            pl.BlockSpec((D, D), lambda i: (0, 0)),
            pl.BlockSpec((1, D), lambda i: (0, 0)),
            pl.BlockSpec((D, D), lambda i: (0, 0)),
            pl.BlockSpec((1, D), lambda i: (0, 0)),
            pl.BlockSpec((D, C), lambda i: (0, 0)),
            pl.BlockSpec((1, C), lambda i: (0, 0)),
        ],
        out_specs=pl.BlockSpec((BLK, C), lambda i: (i, 0)),
        out_shape=jax.ShapeDtypeStruct((P, C), jnp.float32),
    )(accp, hsp, dinv, b3, wf1, bf1, wf2, bf2, wf3, bf3)


def kernel(TRAIN, x, edge_index, W1, b1, W2, b2, W3, b3,
           Wf1, bf1, Wf2, bf2, Wf3, bf3):
    del TRAIN
    src = edge_index[0].astype(jnp.int32)
    dst = edge_index[1].astype(jnp.int32)
    src3 = src.reshape(NW, NCH, CHUNK)
    dst3 = dst.reshape(NW, NCH, CHUNK)

    xp = jnp.zeros((P, D), jnp.float32).at[:N].set(x)
    zeros_pd = jnp.zeros((P, D), jnp.float32)
    ones_c = jnp.ones((CHUNK, D), jnp.float32)

    deg = _deg_call(dst3, ones_c, zeros_pd)
    hs1, dinv = _tc_first(xp, W1, deg)
    acc1 = _spmm_call(hs1, src3, dst3, zeros_pd)
    hs2 = _tc_mid(acc1, hs1, dinv, b1.reshape(1, D), W2)
    acc2 = _spmm_call(hs2, src3, dst3, zeros_pd)
    hs3 = _tc_mid(acc2, hs2, dinv, b2.reshape(1, D), W3)
    acc3 = _spmm_call(hs3, src3, dst3, zeros_pd)
    out = _tc_head(acc3, hs3, dinv, b3.reshape(1, D),
                   Wf1, bf1.reshape(1, D), Wf2, bf2.reshape(1, D),
                   Wf3, bf3.reshape(1, C))
    return out[:N]


# speedup vs baseline: 20.7534x; 1.0789x over previous
import functools

import jax
import jax.numpy as jnp
from jax import lax
from jax.experimental import pallas as pl
from jax.experimental.pallas import tpu as pltpu
from jax.experimental.pallas import tpu_sc as plsc

N = 10000
P = 10240
E = 320000
D = 128
C = 16
NC, NS = 2, 16
NW = NC * NS
CHUNK = 80
EPT = E // NW
NCH = EPT // CHUNK
ROWS_T = P // NS

_mesh = plsc.VectorSubcoreMesh(
    core_axis_name="c", subcore_axis_name="s", num_cores=NC, num_subcores=NS)


DEG_INFLIGHT = 4


def _deg_body(dst3, ones_h, zeros_h, deg_out, didx, ones_v, dacc, ssem):
    cid = lax.axis_index("c")
    sid = lax.axis_index("s")
    wid = cid * NS + sid

    pltpu.sync_copy(dst3.at[wid], didx)
    pltpu.sync_copy(ones_h, ones_v)
    pltpu.sync_copy(zeros_h.at[pl.ds(sid * ROWS_T, ROWS_T)],
                    dacc.at[pl.ds(sid * ROWS_T, ROWS_T)])
    plsc.subcore_barrier()

    def body(j, _):
        pltpu.async_copy(ones_v, dacc.at[didx.at[j]], ssem, add=True)

        @pl.when(j >= DEG_INFLIGHT)
        def _():
            pltpu.make_async_copy(ones_v, dacc.at[didx.at[0]], ssem).wait()

        return 0

    lax.fori_loop(0, NCH, body, 0)

    def drain(j, _):
        pltpu.make_async_copy(ones_v, dacc.at[didx.at[0]], ssem).wait()
        return 0

    lax.fori_loop(0, DEG_INFLIGHT, drain, 0)
    plsc.subcore_barrier()
    pltpu.sync_copy(dacc.at[pl.ds(sid * ROWS_T, ROWS_T)],
                    deg_out.at[cid, pl.ds(sid * ROWS_T, ROWS_T)])


_deg_call = pl.kernel(
    _deg_body,
    out_type=jax.ShapeDtypeStruct((NC, P, D), jnp.float32),
    mesh=_mesh,
    scratch_types=[
        pltpu.VMEM((NCH, CHUNK), jnp.int32),
        pltpu.VMEM((CHUNK, D), jnp.float32),
        pltpu.VMEM_SHARED((P, D), jnp.float32),
        pltpu.SemaphoreType.DMA,
    ],
)


def _spmm_body(hs, src3, dst3, zeros_h, acc_out, sidx, didx, rows, acc,
               gsem, ssem, isem):
    cid = lax.axis_index("c")
    sid = lax.axis_index("s")
    wid = cid * NS + sid

    pltpu.sync_copy(zeros_h.at[pl.ds(sid * ROWS_T, ROWS_T)],
                    acc.at[pl.ds(sid * ROWS_T, ROWS_T)])
    plsc.subcore_barrier()

    pltpu.sync_copy(src3.at[wid, 0], sidx.at[0])
    pltpu.sync_copy(dst3.at[wid, 0], didx.at[0])
    pltpu.sync_copy(src3.at[wid, 1], sidx.at[1])
    pltpu.sync_copy(dst3.at[wid, 1], didx.at[1])
    pltpu.async_copy(hs.at[sidx.at[0]], rows.at[0], gsem)
    pltpu.async_copy(hs.at[sidx.at[1]], rows.at[1], gsem)

    def body(j, _):
        s0 = lax.rem(j, 3)
        s2 = lax.rem(j + 2, 3)

        @pl.when(j >= 1)
        def _():
            pltpu.make_async_copy(rows.at[s2], acc.at[didx.at[s2]],
                                  ssem).wait()

        @pl.when(j + 2 < NCH)
        def _():
            pltpu.async_copy(src3.at[wid, j + 2], sidx.at[s2], isem)
            pltpu.async_copy(dst3.at[wid, j + 2], didx.at[s2], isem)

        pltpu.make_async_copy(hs.at[sidx.at[s0]], rows.at[s0], gsem).wait()

        @pl.when(j + 2 < NCH)
        def _():
            pltpu.make_async_copy(src3.at[wid, j + 2], sidx.at[s2],
                                  isem).wait()
            pltpu.make_async_copy(dst3.at[wid, j + 2], didx.at[s2],
                                  isem).wait()
            pltpu.async_copy(hs.at[sidx.at[s2]], rows.at[s2], gsem)

        pltpu.async_copy(rows.at[s0], acc.at[didx.at[s0]], ssem, add=True)
        return 0

    lax.fori_loop(0, NCH, body, 0)
    last = lax.rem(NCH - 1, 3)
    pltpu.make_async_copy(rows.at[last], acc.at[didx.at[last]], ssem).wait()
    plsc.subcore_barrier()
    pltpu.sync_copy(acc.at[pl.ds(sid * ROWS_T, ROWS_T)],
                    acc_out.at[cid, pl.ds(sid * ROWS_T, ROWS_T)])


_spmm_call = pl.kernel(
    _spmm_body,
    out_type=jax.ShapeDtypeStruct((NC, P, D), jnp.float32),
    mesh=_mesh,
    scratch_types=[
        pltpu.VMEM((3, CHUNK), jnp.int32),
        pltpu.VMEM((3, CHUNK), jnp.int32),
        pltpu.VMEM((3, CHUNK, D), jnp.float32),
        pltpu.VMEM_SHARED((P, D), jnp.float32),
        pltpu.SemaphoreType.DMA,
        pltpu.SemaphoreType.DMA,
        pltpu.SemaphoreType.DMA,
    ],
)


BLK = 1024
GRID = P // BLK


def _first_body(x_ref, w_ref, deg_ref, hs_ref, dinv_ref):
    dinv = lax.rsqrt(deg_ref[0][:, 0:1] + deg_ref[1][:, 0:1] + 1.0)
    h = jnp.dot(x_ref[...], w_ref[...], preferred_element_type=jnp.float32)
    hs_ref[...] = h * dinv
    dinv_ref[...] = dinv


def _tc_first(xp, w1, deg):
    return pl.pallas_call(
        _first_body,
        grid=(GRID,),
        in_specs=[
            pl.BlockSpec((BLK, D), lambda i: (i, 0)),
            pl.BlockSpec((D, D), lambda i: (0, 0)),
            pl.BlockSpec((NC, BLK, D), lambda i: (0, i, 0)),
        ],
        out_specs=[
            pl.BlockSpec((BLK, D), lambda i: (i, 0)),
            pl.BlockSpec((BLK, 1), lambda i: (i, 0)),
        ],
        out_shape=[
            jax.ShapeDtypeStruct((P, D), jnp.float32),
            jax.ShapeDtypeStruct((P, 1), jnp.float32),
        ],
    )(xp, w1, deg)


def _mid_body(acc_ref, hsp_ref, dinv_ref, b_ref, w_ref, out_ref):
    t = acc_ref[0] + acc_ref[1] + hsp_ref[...]
    y = jnp.maximum(t * dinv_ref[...] + b_ref[...], 0.0)
    out_ref[...] = jnp.dot(
        y, w_ref[...], preferred_element_type=jnp.float32) * dinv_ref[...]


def _tc_mid(accp, hsp, dinv, b_prev, w_next):
    return pl.pallas_call(
        _mid_body,
        grid=(GRID,),
        in_specs=[
            pl.BlockSpec((NC, BLK, D), lambda i: (0, i, 0)),
            pl.BlockSpec((BLK, D), lambda i: (i, 0)),
            pl.BlockSpec((BLK, 1), lambda i: (i, 0)),
            pl.BlockSpec((1, D), lambda i: (0, 0)),
            pl.BlockSpec((D, D), lambda i: (0, 0)),
        ],
        out_specs=pl.BlockSpec((BLK, D), lambda i: (i, 0)),
        out_shape=jax.ShapeDtypeStruct((P, D), jnp.float32),
    )(accp, hsp, dinv, b_prev, w_next)


def _head_body(acc_ref, hsp_ref, dinv_ref, b3_ref, wf1_ref, bf1_ref,
               wf2_ref, bf2_ref, wf3_ref, bf3_ref, out_ref):
    t = acc_ref[0] + acc_ref[1] + hsp_ref[...]
    y = jnp.maximum(t * dinv_ref[...] + b3_ref[...], 0.0)
    y = jnp.maximum(
        jnp.dot(y, wf1_ref[...], preferred_element_type=jnp.float32)
        + bf1_ref[...], 0.0)
    y = jnp.maximum(
        jnp.dot(y, wf2_ref[...], preferred_element_type=jnp.float32)
        + bf2_ref[...], 0.0)
    z = jnp.dot(y, wf3_ref[...], preferred_element_type=jnp.float32) + bf3_ref[...]
    z = z - jnp.max(z, axis=1, keepdims=True)
    out_ref[...] = z - jnp.log(jnp.sum(jnp.exp(z), axis=1, keepdims=True))


def _tc_head(accp, hsp, dinv, b3, wf1, bf1, wf2, bf2, wf3, bf3):
    return pl.pallas_call(
        _head_body,
        grid=(GRID,),
        in_specs=[
            pl.BlockSpec((NC, BLK, D), lambda i: (0, i, 0)),
            pl.BlockSpec((BLK, D), lambda i: (i, 0)),
            pl.BlockSpec((BLK, 1), lambda i: (i, 0)),
            pl.BlockSpec((1, D), lambda i: (0, 0)),
            pl.BlockSpec((D, D), lambda i: (0, 0)),
            pl.BlockSpec((1, D), lambda i: (0, 0)),
            pl.BlockSpec((D, D), lambda i: (0, 0)),
            pl.BlockSpec((1, D), lambda i: (0, 0)),
            pl.BlockSpec((D, C), lambda i: (0, 0)),
            pl.BlockSpec((1, C), lambda i: (0, 0)),
        ],
        out_specs=pl.BlockSpec((BLK, C), lambda i: (i, 0)),
        out_shape=jax.ShapeDtypeStruct((P, C), jnp.float32),
    )(accp, hsp, dinv, b3, wf1, bf1, wf2, bf2, wf3, bf3)


def kernel(TRAIN, x, edge_index, W1, b1, W2, b2, W3, b3,
           Wf1, bf1, Wf2, bf2, Wf3, bf3):
    del TRAIN
    src = edge_index[0].astype(jnp.int32)
    dst = edge_index[1].astype(jnp.int32)
    src3 = src.reshape(NW, NCH, CHUNK)
    dst3 = dst.reshape(NW, NCH, CHUNK)

    xp = jnp.zeros((P, D), jnp.float32).at[:N].set(x)
    zeros_pd = jnp.zeros((P, D), jnp.float32)
    ones_c = jnp.ones((CHUNK, D), jnp.float32)

    deg = _deg_call(dst3, ones_c, zeros_pd)
    hs1, dinv = _tc_first(xp, W1, deg)
    acc1 = _spmm_call(hs1, src3, dst3, zeros_pd)
    hs2 = _tc_mid(acc1, hs1, dinv, b1.reshape(1, D), W2)
    acc2 = _spmm_call(hs2, src3, dst3, zeros_pd)
    hs3 = _tc_mid(acc2, hs2, dinv, b2.reshape(1, D), W3)
    acc3 = _spmm_call(hs3, src3, dst3, zeros_pd)
    out = _tc_head(acc3, hs3, dinv, b3.reshape(1, D),
                   Wf1, bf1.reshape(1, D), Wf2, bf2.reshape(1, D),
                   Wf3, bf3.reshape(1, C))
    return out[:N]


# speedup vs baseline: 20.8540x; 1.0049x over previous
import functools

import jax
import jax.numpy as jnp
from jax import lax
from jax.experimental import pallas as pl
from jax.experimental.pallas import tpu as pltpu
from jax.experimental.pallas import tpu_sc as plsc

N = 10000
P = 10240
E = 320000
D = 128
C = 16
NC, NS = 2, 16
NW = NC * NS
CHUNK = 80
EPT = E // NW
NCH = EPT // CHUNK
ROWS_T = P // NS

_mesh = plsc.VectorSubcoreMesh(
    core_axis_name="c", subcore_axis_name="s", num_cores=NC, num_subcores=NS)


DEG_INFLIGHT = 4


def _deg_body(dst3, ones_h, zeros_h, deg_out, didx, ones_v, dacc, ssem):
    cid = lax.axis_index("c")
    sid = lax.axis_index("s")
    wid = cid * NS + sid

    pltpu.sync_copy(dst3.at[wid], didx)
    pltpu.sync_copy(ones_h, ones_v)
    pltpu.sync_copy(zeros_h.at[pl.ds(sid * ROWS_T, ROWS_T)],
                    dacc.at[pl.ds(sid * ROWS_T, ROWS_T)])
    plsc.subcore_barrier()

    def body(j, _):
        pltpu.async_copy(ones_v, dacc.at[didx.at[j]], ssem, add=True)

        @pl.when(j >= DEG_INFLIGHT)
        def _():
            pltpu.make_async_copy(ones_v, dacc.at[didx.at[0]], ssem).wait()

        return 0

    lax.fori_loop(0, NCH, body, 0)

    def drain(j, _):
        pltpu.make_async_copy(ones_v, dacc.at[didx.at[0]], ssem).wait()
        return 0

    lax.fori_loop(0, DEG_INFLIGHT, drain, 0)
    plsc.subcore_barrier()
    pltpu.sync_copy(dacc.at[pl.ds(sid * ROWS_T, ROWS_T)],
                    deg_out.at[cid, pl.ds(sid * ROWS_T, ROWS_T)])


_deg_call = pl.kernel(
    _deg_body,
    out_type=jax.ShapeDtypeStruct((NC, P, D), jnp.float32),
    mesh=_mesh,
    scratch_types=[
        pltpu.VMEM((NCH, CHUNK), jnp.int32),
        pltpu.VMEM((CHUNK, D), jnp.float32),
        pltpu.VMEM_SHARED((P, D), jnp.float32),
        pltpu.SemaphoreType.DMA,
    ],
)


def _spmm_body(hs, src3, dst3, zeros_h, acc_out, sidx, didx, rows, acc,
               gsem, ssem, isem):
    cid = lax.axis_index("c")
    sid = lax.axis_index("s")
    wid = cid * NS + sid

    pltpu.sync_copy(src3.at[wid, 0], sidx.at[0])
    pltpu.sync_copy(dst3.at[wid, 0], didx.at[0])
    pltpu.sync_copy(src3.at[wid, 1], sidx.at[1])
    pltpu.sync_copy(dst3.at[wid, 1], didx.at[1])
    pltpu.async_copy(hs.at[sidx.at[0]], rows.at[0], gsem)
    pltpu.async_copy(hs.at[sidx.at[1]], rows.at[1], gsem)

    pltpu.sync_copy(zeros_h.at[pl.ds(sid * ROWS_T, ROWS_T)],
                    acc.at[pl.ds(sid * ROWS_T, ROWS_T)])
    plsc.subcore_barrier()

    def body(j, _):
        s0 = lax.rem(j, 3)
        s2 = lax.rem(j + 2, 3)

        @pl.when(j >= 1)
        def _():
            pltpu.make_async_copy(rows.at[s2], acc.at[didx.at[s2]],
                                  ssem).wait()

        @pl.when(j + 2 < NCH)
        def _():
            pltpu.async_copy(src3.at[wid, j + 2], sidx.at[s2], isem)
            pltpu.async_copy(dst3.at[wid, j + 2], didx.at[s2], isem)

        pltpu.make_async_copy(hs.at[sidx.at[s0]], rows.at[s0], gsem).wait()

        @pl.when(j + 2 < NCH)
        def _():
            pltpu.make_async_copy(src3.at[wid, j + 2], sidx.at[s2],
                                  isem).wait()
            pltpu.make_async_copy(dst3.at[wid, j + 2], didx.at[s2],
                                  isem).wait()
            pltpu.async_copy(hs.at[sidx.at[s2]], rows.at[s2], gsem)

        pltpu.async_copy(rows.at[s0], acc.at[didx.at[s0]], ssem, add=True)
        return 0

    lax.fori_loop(0, NCH, body, 0)
    last = lax.rem(NCH - 1, 3)
    pltpu.make_async_copy(rows.at[last], acc.at[didx.at[last]], ssem).wait()
    plsc.subcore_barrier()
    pltpu.sync_copy(acc.at[pl.ds(sid * ROWS_T, ROWS_T)],
                    acc_out.at[cid, pl.ds(sid * ROWS_T, ROWS_T)])


_spmm_call = pl.kernel(
    _spmm_body,
    out_type=jax.ShapeDtypeStruct((NC, P, D), jnp.float32),
    mesh=_mesh,
    scratch_types=[
        pltpu.VMEM((3, CHUNK), jnp.int32),
        pltpu.VMEM((3, CHUNK), jnp.int32),
        pltpu.VMEM((3, CHUNK, D), jnp.float32),
        pltpu.VMEM_SHARED((P, D), jnp.float32),
        pltpu.SemaphoreType.DMA,
        pltpu.SemaphoreType.DMA,
        pltpu.SemaphoreType.DMA,
    ],
)


BLK = 1024
GRID = P // BLK


def _first_body(x_ref, w_ref, deg_ref, hs_ref, dinv_ref):
    dinv = lax.rsqrt(deg_ref[0][:, 0:1] + deg_ref[1][:, 0:1] + 1.0)
    h = jnp.dot(x_ref[...], w_ref[...], preferred_element_type=jnp.float32)
    hs_ref[...] = h * dinv
    dinv_ref[...] = dinv


def _tc_first(xp, w1, deg):
    return pl.pallas_call(
        _first_body,
        grid=(GRID,),
        in_specs=[
            pl.BlockSpec((BLK, D), lambda i: (i, 0)),
            pl.BlockSpec((D, D), lambda i: (0, 0)),
            pl.BlockSpec((NC, BLK, D), lambda i: (0, i, 0)),
        ],
        out_specs=[
            pl.BlockSpec((BLK, D), lambda i: (i, 0)),
            pl.BlockSpec((BLK, 1), lambda i: (i, 0)),
        ],
        out_shape=[
            jax.ShapeDtypeStruct((P, D), jnp.float32),
            jax.ShapeDtypeStruct((P, 1), jnp.float32),
        ],
    )(xp, w1, deg)


def _mid_body(acc_ref, hsp_ref, dinv_ref, b_ref, w_ref, out_ref):
    t = acc_ref[0] + acc_ref[1] + hsp_ref[...]
    y = jnp.maximum(t * dinv_ref[...] + b_ref[...], 0.0)
    out_ref[...] = jnp.dot(
        y, w_ref[...], preferred_element_type=jnp.float32) * dinv_ref[...]


def _tc_mid(accp, hsp, dinv, b_prev, w_next):
    return pl.pallas_call(
        _mid_body,
        grid=(GRID,),
        in_specs=[
            pl.BlockSpec((NC, BLK, D), lambda i: (0, i, 0)),
            pl.BlockSpec((BLK, D), lambda i: (i, 0)),
            pl.BlockSpec((BLK, 1), lambda i: (i, 0)),
            pl.BlockSpec((1, D), lambda i: (0, 0)),
            pl.BlockSpec((D, D), lambda i: (0, 0)),
        ],
        out_specs=pl.BlockSpec((BLK, D), lambda i: (i, 0)),
        out_shape=jax.ShapeDtypeStruct((P, D), jnp.float32),
    )(accp, hsp, dinv, b_prev, w_next)


def _head_body(acc_ref, hsp_ref, dinv_ref, b3_ref, wf1_ref, bf1_ref,
               wf2_ref, bf2_ref, wf3_ref, bf3_ref, out_ref):
    t = acc_ref[0] + acc_ref[1] + hsp_ref[...]
    y = jnp.maximum(t * dinv_ref[...] + b3_ref[...], 0.0)
    y = jnp.maximum(
        jnp.dot(y, wf1_ref[...], preferred_element_type=jnp.float32)
        + bf1_ref[...], 0.0)
    y = jnp.maximum(
        jnp.dot(y, wf2_ref[...], preferred_element_type=jnp.float32)
        + bf2_ref[...], 0.0)
    z = jnp.dot(y, wf3_ref[...], preferred_element_type=jnp.float32) + bf3_ref[...]
    z = z - jnp.max(z, axis=1, keepdims=True)
    out_ref[...] = z - jnp.log(jnp.sum(jnp.exp(z), axis=1, keepdims=True))


def _tc_head(accp, hsp, dinv, b3, wf1, bf1, wf2, bf2, wf3, bf3):
    return pl.pallas_call(
        _head_body,
        grid=(GRID,),
        in_specs=[
            pl.BlockSpec((NC, BLK, D), lambda i: (0, i, 0)),
            pl.BlockSpec((BLK, D), lambda i: (i, 0)),
            pl.BlockSpec((BLK, 1), lambda i: (i, 0)),
            pl.BlockSpec((1, D), lambda i: (0, 0)),
            pl.BlockSpec((D, D), lambda i: (0, 0)),
            pl.BlockSpec((1, D), lambda i: (0, 0)),
            pl.BlockSpec((D, D), lambda i: (0, 0)),
            pl.BlockSpec((1, D), lambda i: (0, 0)),
            pl.BlockSpec((D, C), lambda i: (0, 0)),
            pl.BlockSpec((1, C), lambda i: (0, 0)),
        ],
        out_specs=pl.BlockSpec((BLK, C), lambda i: (i, 0)),
        out_shape=jax.ShapeDtypeStruct((P, C), jnp.float32),
    )(accp, hsp, dinv, b3, wf1, bf1, wf2, bf2, wf3, bf3)


def kernel(TRAIN, x, edge_index, W1, b1, W2, b2, W3, b3,
           Wf1, bf1, Wf2, bf2, Wf3, bf3):
    del TRAIN
    src = edge_index[0].astype(jnp.int32)
    dst = edge_index[1].astype(jnp.int32)
    src3 = src.reshape(NW, NCH, CHUNK)
    dst3 = dst.reshape(NW, NCH, CHUNK)

    xp = jnp.zeros((P, D), jnp.float32).at[:N].set(x)
    zeros_pd = jnp.zeros((P, D), jnp.float32)
    ones_c = jnp.ones((CHUNK, D), jnp.float32)

    deg = _deg_call(dst3, ones_c, zeros_pd)
    hs1, dinv = _tc_first(xp, W1, deg)
    acc1 = _spmm_call(hs1, src3, dst3, zeros_pd)
    hs2 = _tc_mid(acc1, hs1, dinv, b1.reshape(1, D), W2)
    acc2 = _spmm_call(hs2, src3, dst3, zeros_pd)
    hs3 = _tc_mid(acc2, hs2, dinv, b2.reshape(1, D), W3)
    acc3 = _spmm_call(hs3, src3, dst3, zeros_pd)
    out = _tc_head(acc3, hs3, dinv, b3.reshape(1, D),
                   Wf1, bf1.reshape(1, D), Wf2, bf2.reshape(1, D),
                   Wf3, bf3.reshape(1, C))
    return out[:N]


# speedup vs baseline: 20.9433x; 1.0043x over previous
import functools

import jax
import jax.numpy as jnp
from jax import lax
from jax.experimental import pallas as pl
from jax.experimental.pallas import tpu as pltpu
from jax.experimental.pallas import tpu_sc as plsc

N = 10000
P = 10240
E = 320000
D = 128
C = 16
NC, NS = 2, 16
NW = NC * NS
CHUNK = 80
EPT = E // NW
NCH = EPT // CHUNK
ROWS_T = P // NS

_mesh = plsc.VectorSubcoreMesh(
    core_axis_name="c", subcore_axis_name="s", num_cores=NC, num_subcores=NS)


DEG_INFLIGHT = 8


def _deg_body(dst3, ones_h, zeros_h, deg_out, didx, ones_v, dacc, ssem):
    cid = lax.axis_index("c")
    sid = lax.axis_index("s")
    wid = cid * NS + sid

    pltpu.async_copy(dst3.at[wid], didx, ssem)
    pltpu.async_copy(ones_h, ones_v, ssem)
    pltpu.sync_copy(zeros_h.at[pl.ds(sid * ROWS_T, ROWS_T)],
                    dacc.at[pl.ds(sid * ROWS_T, ROWS_T)])
    pltpu.make_async_copy(dst3.at[wid], didx, ssem).wait()
    pltpu.make_async_copy(ones_h, ones_v, ssem).wait()
    plsc.subcore_barrier()

    def body(j, _):
        pltpu.async_copy(ones_v, dacc.at[didx.at[j]], ssem, add=True)

        @pl.when(j >= DEG_INFLIGHT)
        def _():
            pltpu.make_async_copy(ones_v, dacc.at[didx.at[0]], ssem).wait()

        return 0

    lax.fori_loop(0, NCH, body, 0)

    def drain(j, _):
        pltpu.make_async_copy(ones_v, dacc.at[didx.at[0]], ssem).wait()
        return 0

    lax.fori_loop(0, DEG_INFLIGHT, drain, 0)
    plsc.subcore_barrier()
    pltpu.sync_copy(dacc.at[pl.ds(sid * ROWS_T, ROWS_T)],
                    deg_out.at[cid, pl.ds(sid * ROWS_T, ROWS_T)])


_deg_call = pl.kernel(
    _deg_body,
    out_type=jax.ShapeDtypeStruct((NC, P, D), jnp.float32),
    mesh=_mesh,
    scratch_types=[
        pltpu.VMEM((NCH, CHUNK), jnp.int32),
        pltpu.VMEM((CHUNK, D), jnp.float32),
        pltpu.VMEM_SHARED((P, D), jnp.float32),
        pltpu.SemaphoreType.DMA,
    ],
)


def _spmm_body(hs, src3, dst3, zeros_h, acc_out, sidx, didx, rows, acc,
               gsem, ssem, isem):
    cid = lax.axis_index("c")
    sid = lax.axis_index("s")
    wid = cid * NS + sid

    pltpu.sync_copy(src3.at[wid, 0], sidx.at[0])
    pltpu.sync_copy(dst3.at[wid, 0], didx.at[0])
    pltpu.sync_copy(src3.at[wid, 1], sidx.at[1])
    pltpu.sync_copy(dst3.at[wid, 1], didx.at[1])
    pltpu.async_copy(hs.at[sidx.at[0]], rows.at[0], gsem)
    pltpu.async_copy(hs.at[sidx.at[1]], rows.at[1], gsem)

    pltpu.sync_copy(zeros_h.at[pl.ds(sid * ROWS_T, ROWS_T)],
                    acc.at[pl.ds(sid * ROWS_T, ROWS_T)])
    plsc.subcore_barrier()

    def body(j, _):
        s0 = lax.rem(j, 3)
        s2 = lax.rem(j + 2, 3)

        @pl.when(j >= 1)
        def _():
            pltpu.make_async_copy(rows.at[s2], acc.at[didx.at[s2]],
                                  ssem).wait()

        @pl.when(j + 2 < NCH)
        def _():
            pltpu.async_copy(src3.at[wid, j + 2], sidx.at[s2], isem)
            pltpu.async_copy(dst3.at[wid, j + 2], didx.at[s2], isem)

        pltpu.make_async_copy(hs.at[sidx.at[s0]], rows.at[s0], gsem).wait()

        @pl.when(j + 2 < NCH)
        def _():
            pltpu.make_async_copy(src3.at[wid, j + 2], sidx.at[s2],
                                  isem).wait()
            pltpu.make_async_copy(dst3.at[wid, j + 2], didx.at[s2],
                                  isem).wait()
            pltpu.async_copy(hs.at[sidx.at[s2]], rows.at[s2], gsem)

        pltpu.async_copy(rows.at[s0], acc.at[didx.at[s0]], ssem, add=True)
        return 0

    lax.fori_loop(0, NCH, body, 0)
    last = lax.rem(NCH - 1, 3)
    pltpu.make_async_copy(rows.at[last], acc.at[didx.at[last]], ssem).wait()
    plsc.subcore_barrier()
    pltpu.sync_copy(acc.at[pl.ds(sid * ROWS_T, ROWS_T)],
                    acc_out.at[cid, pl.ds(sid * ROWS_T, ROWS_T)])


_spmm_call = pl.kernel(
    _spmm_body,
    out_type=jax.ShapeDtypeStruct((NC, P, D), jnp.float32),
    mesh=_mesh,
    scratch_types=[
        pltpu.VMEM((3, CHUNK), jnp.int32),
        pltpu.VMEM((3, CHUNK), jnp.int32),
        pltpu.VMEM((3, CHUNK, D), jnp.float32),
        pltpu.VMEM_SHARED((P, D), jnp.float32),
        pltpu.SemaphoreType.DMA,
        pltpu.SemaphoreType.DMA,
        pltpu.SemaphoreType.DMA,
    ],
)


BLK = 1024
GRID = P // BLK


def _first_body(x_ref, w_ref, deg_ref, hs_ref, dinv_ref):
    dinv = lax.rsqrt(deg_ref[0][:, 0:1] + deg_ref[1][:, 0:1] + 1.0)
    h = jnp.dot(x_ref[...], w_ref[...], preferred_element_type=jnp.float32)
    hs_ref[...] = h * dinv
    dinv_ref[...] = dinv


def _tc_first(xp, w1, deg):
    return pl.pallas_call(
        _first_body,
        grid=(GRID,),
        in_specs=[
            pl.BlockSpec((BLK, D), lambda i: (i, 0)),
            pl.BlockSpec((D, D), lambda i: (0, 0)),
            pl.BlockSpec((NC, BLK, D), lambda i: (0, i, 0)),
        ],
        out_specs=[
            pl.BlockSpec((BLK, D), lambda i: (i, 0)),
            pl.BlockSpec((BLK, 1), lambda i: (i, 0)),
        ],
        out_shape=[
            jax.ShapeDtypeStruct((P, D), jnp.float32),
            jax.ShapeDtypeStruct((P, 1), jnp.float32),
        ],
    )(xp, w1, deg)


def _mid_body(acc_ref, hsp_ref, dinv_ref, b_ref, w_ref, out_ref):
    t = acc_ref[0] + acc_ref[1] + hsp_ref[...]
    y = jnp.maximum(t * dinv_ref[...] + b_ref[...], 0.0)
    out_ref[...] = jnp.dot(
        y, w_ref[...], preferred_element_type=jnp.float32) * dinv_ref[...]


def _tc_mid(accp, hsp, dinv, b_prev, w_next):
    return pl.pallas_call(
        _mid_body,
        grid=(GRID,),
        in_specs=[
            pl.BlockSpec((NC, BLK, D), lambda i: (0, i, 0)),
            pl.BlockSpec((BLK, D), lambda i: (i, 0)),
            pl.BlockSpec((BLK, 1), lambda i: (i, 0)),
            pl.BlockSpec((1, D), lambda i: (0, 0)),
            pl.BlockSpec((D, D), lambda i: (0, 0)),
        ],
        out_specs=pl.BlockSpec((BLK, D), lambda i: (i, 0)),
        out_shape=jax.ShapeDtypeStruct((P, D), jnp.float32),
    )(accp, hsp, dinv, b_prev, w_next)


def _head_body(acc_ref, hsp_ref, dinv_ref, b3_ref, wf1_ref, bf1_ref,
               wf2_ref, bf2_ref, wf3_ref, bf3_ref, out_ref):
    t = acc_ref[0] + acc_ref[1] + hsp_ref[...]
    y = jnp.maximum(t * dinv_ref[...] + b3_ref[...], 0.0)
    y = jnp.maximum(
        jnp.dot(y, wf1_ref[...], preferred_element_type=jnp.float32)
        + bf1_ref[...], 0.0)
    y = jnp.maximum(
        jnp.dot(y, wf2_ref[...], preferred_element_type=jnp.float32)
        + bf2_ref[...], 0.0)
    z = jnp.dot(y, wf3_ref[...], preferred_element_type=jnp.float32) + bf3_ref[...]
    z = z - jnp.max(z, axis=1, keepdims=True)
    out_ref[...] = z - jnp.log(jnp.sum(jnp.exp(z), axis=1, keepdims=True))


def _tc_head(accp, hsp, dinv, b3, wf1, bf1, wf2, bf2, wf3, bf3):
    return pl.pallas_call(
        _head_body,
        grid=(GRID,),
        in_specs=[
            pl.BlockSpec((NC, BLK, D), lambda i: (0, i, 0)),
            pl.BlockSpec((BLK, D), lambda i: (i, 0)),
            pl.BlockSpec((BLK, 1), lambda i: (i, 0)),
            pl.BlockSpec((1, D), lambda i: (0, 0)),
            pl.BlockSpec((D, D), lambda i: (0, 0)),
            pl.BlockSpec((1, D), lambda i: (0, 0)),
            pl.BlockSpec((D, D), lambda i: (0, 0)),
            pl.BlockSpec((1, D), lambda i: (0, 0)),
            pl.BlockSpec((D, C), lambda i: (0, 0)),
            pl.BlockSpec((1, C), lambda i: (0, 0)),
        ],
        out_specs=pl.BlockSpec((BLK, C), lambda i: (i, 0)),
        out_shape=jax.ShapeDtypeStruct((P, C), jnp.float32),
    )(accp, hsp, dinv, b3, wf1, bf1, wf2, bf2, wf3, bf3)


def kernel(TRAIN, x, edge_index, W1, b1, W2, b2, W3, b3,
           Wf1, bf1, Wf2, bf2, Wf3, bf3):
    del TRAIN
    src = edge_index[0].astype(jnp.int32)
    dst = edge_index[1].astype(jnp.int32)
    src3 = src.reshape(NW, NCH, CHUNK)
    dst3 = dst.reshape(NW, NCH, CHUNK)

    xp = jnp.zeros((P, D), jnp.float32).at[:N].set(x)
    zeros_pd = jnp.zeros((P, D), jnp.float32)
    ones_c = jnp.ones((CHUNK, D), jnp.float32)

    deg = _deg_call(dst3, ones_c, zeros_pd)
    hs1, dinv = _tc_first(xp, W1, deg)
    acc1 = _spmm_call(hs1, src3, dst3, zeros_pd)
    hs2 = _tc_mid(acc1, hs1, dinv, b1.reshape(1, D), W2)
    acc2 = _spmm_call(hs2, src3, dst3, zeros_pd)
    hs3 = _tc_mid(acc2, hs2, dinv, b2.reshape(1, D), W3)
    acc3 = _spmm_call(hs3, src3, dst3, zeros_pd)
    out = _tc_head(acc3, hs3, dinv, b3.reshape(1, D),
                   Wf1, bf1.reshape(1, D), Wf2, bf2.reshape(1, D),
                   Wf3, bf3.reshape(1, C))
    return out[:N]


# speedup vs baseline: 20.9574x; 1.0007x over previous
import jax
import jax.numpy as jnp
from jax import lax
from jax.experimental import pallas as pl
from jax.experimental.pallas import tpu as pltpu
from jax.experimental.pallas import tpu_sc as plsc

N = 10000
P = 10240
E = 320000
D = 128
C = 16
NC, NS = 2, 16
NW = NC * NS
CHUNK = 80
EPT = E // NW
NCH = EPT // CHUNK
ROWS_T = P // NS

_mesh = plsc.VectorSubcoreMesh(
    core_axis_name="c", subcore_axis_name="s", num_cores=NC, num_subcores=NS)


DEG_INFLIGHT = 8


def _deg_body(dst3, ones_h, zeros_h, deg_out, didx, ones_v, dacc, ssem):
    cid = lax.axis_index("c")
    sid = lax.axis_index("s")
    wid = cid * NS + sid

    pltpu.async_copy(dst3.at[wid], didx, ssem)
    pltpu.async_copy(ones_h, ones_v, ssem)
    pltpu.sync_copy(zeros_h.at[pl.ds(sid * ROWS_T, ROWS_T)],
                    dacc.at[pl.ds(sid * ROWS_T, ROWS_T)])
    pltpu.make_async_copy(dst3.at[wid], didx, ssem).wait()
    pltpu.make_async_copy(ones_h, ones_v, ssem).wait()
    plsc.subcore_barrier()

    def body(j, _):
        pltpu.async_copy(ones_v, dacc.at[didx.at[j]], ssem, add=True)

        @pl.when(j >= DEG_INFLIGHT)
        def _():
            pltpu.make_async_copy(ones_v, dacc.at[didx.at[0]], ssem).wait()

        return 0

    lax.fori_loop(0, NCH, body, 0)

    def drain(j, _):
        pltpu.make_async_copy(ones_v, dacc.at[didx.at[0]], ssem).wait()
        return 0

    lax.fori_loop(0, DEG_INFLIGHT, drain, 0)
    plsc.subcore_barrier()
    pltpu.sync_copy(dacc.at[pl.ds(sid * ROWS_T, ROWS_T)],
                    deg_out.at[cid, pl.ds(sid * ROWS_T, ROWS_T)])


_deg_call = pl.kernel(
    _deg_body,
    out_type=jax.ShapeDtypeStruct((NC, P, D), jnp.float32),
    mesh=_mesh,
    scratch_types=[
        pltpu.VMEM((NCH, CHUNK), jnp.int32),
        pltpu.VMEM((CHUNK, D), jnp.float32),
        pltpu.VMEM_SHARED((P, D), jnp.float32),
        pltpu.SemaphoreType.DMA,
    ],
)


def _spmm_body(hs, src3, dst3, zeros_h, acc_out, sidx, didx, rows, acc,
               gsem, ssem, isem):
    cid = lax.axis_index("c")
    sid = lax.axis_index("s")
    wid = cid * NS + sid

    pltpu.sync_copy(src3.at[wid, 0], sidx.at[0])
    pltpu.sync_copy(dst3.at[wid, 0], didx.at[0])
    pltpu.sync_copy(src3.at[wid, 1], sidx.at[1])
    pltpu.sync_copy(dst3.at[wid, 1], didx.at[1])
    pltpu.async_copy(hs.at[sidx.at[0]], rows.at[0], gsem)
    pltpu.async_copy(hs.at[sidx.at[1]], rows.at[1], gsem)

    pltpu.sync_copy(zeros_h.at[pl.ds(sid * ROWS_T, ROWS_T)],
                    acc.at[pl.ds(sid * ROWS_T, ROWS_T)])
    plsc.subcore_barrier()

    def body(j, _):
        s0 = lax.rem(j, 3)
        s2 = lax.rem(j + 2, 3)

        @pl.when(j >= 1)
        def _():
            pltpu.make_async_copy(rows.at[s2], acc.at[didx.at[s2]],
                                  ssem).wait()

        @pl.when(j + 2 < NCH)
        def _():
            pltpu.async_copy(src3.at[wid, j + 2], sidx.at[s2], isem)
            pltpu.async_copy(dst3.at[wid, j + 2], didx.at[s2], isem)

        pltpu.make_async_copy(hs.at[sidx.at[s0]], rows.at[s0], gsem).wait()

        @pl.when(j + 2 < NCH)
        def _():
            pltpu.make_async_copy(src3.at[wid, j + 2], sidx.at[s2],
                                  isem).wait()
            pltpu.make_async_copy(dst3.at[wid, j + 2], didx.at[s2],
                                  isem).wait()
            pltpu.async_copy(hs.at[sidx.at[s2]], rows.at[s2], gsem)

        pltpu.async_copy(rows.at[s0], acc.at[didx.at[s0]], ssem, add=True)
        return 0

    lax.fori_loop(0, NCH, body, 0)
    last = lax.rem(NCH - 1, 3)
    pltpu.make_async_copy(rows.at[last], acc.at[didx.at[last]], ssem).wait()
    plsc.subcore_barrier()
    pltpu.sync_copy(acc.at[pl.ds(sid * ROWS_T, ROWS_T)],
                    acc_out.at[cid, pl.ds(sid * ROWS_T, ROWS_T)])


_spmm_call = pl.kernel(
    _spmm_body,
    out_type=jax.ShapeDtypeStruct((NC, P, D), jnp.float32),
    mesh=_mesh,
    scratch_types=[
        pltpu.VMEM((3, CHUNK), jnp.int32),
        pltpu.VMEM((3, CHUNK), jnp.int32),
        pltpu.VMEM((3, CHUNK, D), jnp.float32),
        pltpu.VMEM_SHARED((P, D), jnp.float32),
        pltpu.SemaphoreType.DMA,
        pltpu.SemaphoreType.DMA,
        pltpu.SemaphoreType.DMA,
    ],
)


BLK = 1024
GRID = P // BLK


def _first_body(x_ref, w_ref, deg_ref, hs_ref, dinv_ref):
    dinv = lax.rsqrt(deg_ref[0][:, 0:1] + deg_ref[1][:, 0:1] + 1.0)
    h = jnp.dot(x_ref[...], w_ref[...], preferred_element_type=jnp.float32)
    hs_ref[...] = h * dinv
    dinv_ref[...] = dinv


def _tc_first(xp, w1, deg):
    return pl.pallas_call(
        _first_body,
        grid=(GRID,),
        in_specs=[
            pl.BlockSpec((BLK, D), lambda i: (i, 0)),
            pl.BlockSpec((D, D), lambda i: (0, 0)),
            pl.BlockSpec((NC, BLK, D), lambda i: (0, i, 0)),
        ],
        out_specs=[
            pl.BlockSpec((BLK, D), lambda i: (i, 0)),
            pl.BlockSpec((BLK, 1), lambda i: (i, 0)),
        ],
        out_shape=[
            jax.ShapeDtypeStruct((P, D), jnp.float32),
            jax.ShapeDtypeStruct((P, 1), jnp.float32),
        ],
    )(xp, w1, deg)


def _mid_body(acc_ref, hsp_ref, dinv_ref, b_ref, w_ref, out_ref):
    t = acc_ref[0] + acc_ref[1] + hsp_ref[...]
    y = jnp.maximum(t * dinv_ref[...] + b_ref[...], 0.0)
    out_ref[...] = jnp.dot(
        y, w_ref[...], preferred_element_type=jnp.float32) * dinv_ref[...]


def _tc_mid(accp, hsp, dinv, b_prev, w_next):
    return pl.pallas_call(
        _mid_body,
        grid=(GRID,),
        in_specs=[
            pl.BlockSpec((NC, BLK, D), lambda i: (0, i, 0)),
            pl.BlockSpec((BLK, D), lambda i: (i, 0)),
            pl.BlockSpec((BLK, 1), lambda i: (i, 0)),
            pl.BlockSpec((1, D), lambda i: (0, 0)),
            pl.BlockSpec((D, D), lambda i: (0, 0)),
        ],
        out_specs=pl.BlockSpec((BLK, D), lambda i: (i, 0)),
        out_shape=jax.ShapeDtypeStruct((P, D), jnp.float32),
    )(accp, hsp, dinv, b_prev, w_next)


def _head_body(acc_ref, hsp_ref, dinv_ref, b3_ref, wf1_ref, bf1_ref,
               wf2_ref, bf2_ref, wf3_ref, bf3_ref, out_ref):
    t = acc_ref[0] + acc_ref[1] + hsp_ref[...]
    y = jnp.maximum(t * dinv_ref[...] + b3_ref[...], 0.0)
    y = jnp.maximum(
        jnp.dot(y, wf1_ref[...], preferred_element_type=jnp.float32)
        + bf1_ref[...], 0.0)
    y = jnp.maximum(
        jnp.dot(y, wf2_ref[...], preferred_element_type=jnp.float32)
        + bf2_ref[...], 0.0)
    z = jnp.dot(y, wf3_ref[...], preferred_element_type=jnp.float32) + bf3_ref[...]
    z = z - jnp.max(z, axis=1, keepdims=True)
    out_ref[...] = z - jnp.log(jnp.sum(jnp.exp(z), axis=1, keepdims=True))


def _tc_head(accp, hsp, dinv, b3, wf1, bf1, wf2, bf2, wf3, bf3):
    return pl.pallas_call(
        _head_body,
        grid=(GRID,),
        in_specs=[
            pl.BlockSpec((NC, BLK, D), lambda i: (0, i, 0)),
            pl.BlockSpec((BLK, D), lambda i: (i, 0)),
            pl.BlockSpec((BLK, 1), lambda i: (i, 0)),
            pl.BlockSpec((1, D), lambda i: (0, 0)),
            pl.BlockSpec((D, D), lambda i: (0, 0)),
            pl.BlockSpec((1, D), lambda i: (0, 0)),
            pl.BlockSpec((D, D), lambda i: (0, 0)),
            pl.BlockSpec((1, D), lambda i: (0, 0)),
            pl.BlockSpec((D, C), lambda i: (0, 0)),
            pl.BlockSpec((1, C), lambda i: (0, 0)),
        ],
        out_specs=pl.BlockSpec((BLK, C), lambda i: (i, 0)),
        out_shape=jax.ShapeDtypeStruct((P, C), jnp.float32),
    )(accp, hsp, dinv, b3, wf1, bf1, wf2, bf2, wf3, bf3)


def kernel(TRAIN, x, edge_index, W1, b1, W2, b2, W3, b3,
           Wf1, bf1, Wf2, bf2, Wf3, bf3):
    del TRAIN
    src = edge_index[0].astype(jnp.int32)
    dst = edge_index[1].astype(jnp.int32)
    src3 = src.reshape(NW, NCH, CHUNK)
    dst3 = dst.reshape(NW, NCH, CHUNK)

    xp = jnp.zeros((P, D), jnp.float32).at[:N].set(x)
    zeros_pd = jnp.zeros((P, D), jnp.float32)
    ones_c = jnp.ones((CHUNK, D), jnp.float32)

    deg = _deg_call(dst3, ones_c, zeros_pd)
    hs1, dinv = _tc_first(xp, W1, deg)
    acc1 = _spmm_call(hs1, src3, dst3, zeros_pd)
    hs2 = _tc_mid(acc1, hs1, dinv, b1.reshape(1, D), W2)
    acc2 = _spmm_call(hs2, src3, dst3, zeros_pd)
    hs3 = _tc_mid(acc2, hs2, dinv, b2.reshape(1, D), W3)
    acc3 = _spmm_call(hs3, src3, dst3, zeros_pd)
    out = _tc_head(acc3, hs3, dinv, b3.reshape(1, D),
                   Wf1, bf1.reshape(1, D), Wf2, bf2.reshape(1, D),
                   Wf3, bf3.reshape(1, C))
    return out[:N]
